# baseline (speedup 1.0000x reference)
import functools

import jax
import jax.numpy as jnp
from jax import lax
from jax.experimental import pallas as pl
from jax.experimental.pallas import tpu as pltpu
from jax.experimental.pallas import tpu_sc as plsc

D_MODEL = 64
SCALE = 8.0

_NUM_CORES = 2
_NUM_SUBCORES = 16
_NW = _NUM_CORES * _NUM_SUBCORES
_CHUNK = 512


def _emb_body(n_chunks, b_per_w, x_hbm, tab_hbm, out_hbm, idx_v, rows_v, sem):
    wid = lax.axis_index("s") * _NUM_CORES + lax.axis_index("c")
    base = wid * b_per_w

    @pl.loop(0, n_chunks)
    def chunk_loop(g):
        off = base + g * _CHUNK
        pltpu.sync_copy(x_hbm.at[pl.ds(off, _CHUNK)], idx_v)
        pltpu.async_copy(tab_hbm.at[idx_v], rows_v, sem).wait()

        @pl.loop(0, _CHUNK, unroll=4)
        def scale_loop(i):
            for k in range(D_MODEL // 16):
                sl = pl.ds(k * 16, 16)
                rows_v[i, sl] = rows_v[i, sl] * SCALE

        pltpu.sync_copy(rows_v, out_hbm.at[pl.ds(off, _CHUNK)])


def kernel(x, table):
    s0, s1 = x.shape
    b = s0 * s1
    xf = x.reshape(b)
    b_per_w = b // _NW
    n_chunks = b_per_w // _CHUNK
    mesh = plsc.VectorSubcoreMesh(
        core_axis_name="c", subcore_axis_name="s",
        num_cores=_NUM_CORES, num_subcores=_NUM_SUBCORES)
    out = pl.kernel(
        functools.partial(_emb_body, n_chunks, b_per_w),
        out_type=jax.ShapeDtypeStruct((b, D_MODEL), jnp.float32),
        mesh=mesh,
        scratch_types=[
            pltpu.VMEM((_CHUNK,), jnp.int32),
            pltpu.VMEM((_CHUNK, D_MODEL), jnp.float32),
            pltpu.SemaphoreType.DMA,
        ],
        compiler_params=pltpu.CompilerParams(use_tc_tiling_on_sc=False),
    )(xf, table)
    return out.reshape(s0, s1, D_MODEL)


# speedup vs baseline: 1.1474x; 1.1474x over previous
import functools

import jax
import jax.numpy as jnp
from jax import lax
from jax.experimental import pallas as pl
from jax.experimental.pallas import tpu as pltpu
from jax.experimental.pallas import tpu_sc as plsc

D_MODEL = 64
SCALE = 8.0

_NUM_CORES = 2
_NUM_SUBCORES = 16
_NW = _NUM_CORES * _NUM_SUBCORES
_CHUNK = 512


def _emb_body(n_chunks, b_per_w, x_hbm, tab_hbm, out_hbm,
              idx_a, idx_b, rows_a, rows_b, gsem_a, gsem_b, isem_a, isem_b):
    wid = lax.axis_index("s") * _NUM_CORES + lax.axis_index("c")
    base = wid * b_per_w

    def off(g):
        return base + g * _CHUNK

    def scale(rows):
        @pl.loop(0, _CHUNK, unroll=8)
        def _(i):
            for k in range(D_MODEL // 16):
                sl = pl.ds(k * 16, 16)
                rows[i, sl] = rows[i, sl] * SCALE

    pltpu.sync_copy(x_hbm.at[pl.ds(off(0), _CHUNK)], idx_a)
    pltpu.async_copy(tab_hbm.at[idx_a], rows_a, gsem_a)
    pltpu.async_copy(x_hbm.at[pl.ds(1 * _CHUNK + base, _CHUNK)], idx_b, isem_b)

    @pl.loop(0, n_chunks, step=2)
    def chunk_loop(g):
        bufs = (
            (idx_a, rows_a, gsem_a, isem_a, idx_b, rows_b, gsem_b, isem_b),
            (idx_b, rows_b, gsem_b, isem_b, idx_a, rows_a, gsem_a, isem_a),
        )
        for j, (idx_c, rows_c, gsem_c, isem_c,
                idx_o, rows_o, gsem_o, isem_o) in enumerate(bufs):
            cg = g + j
            pltpu.make_async_copy(tab_hbm.at[idx_c], rows_c, gsem_c).wait()

            @pl.when(cg + 2 < n_chunks)
            def _():
                pltpu.async_copy(
                    x_hbm.at[pl.ds(off(cg + 2), _CHUNK)], idx_c, isem_c)

            @pl.when(cg + 1 < n_chunks)
            def _():
                pltpu.make_async_copy(
                    x_hbm.at[pl.ds(off(cg + 1), _CHUNK)], idx_o, isem_o).wait()
                pltpu.async_copy(tab_hbm.at[idx_o], rows_o, gsem_o)

            scale(rows_c)
            pltpu.sync_copy(rows_c, out_hbm.at[pl.ds(off(cg), _CHUNK)])


def kernel(x, table):
    s0, s1 = x.shape
    b = s0 * s1
    xf = x.reshape(b)
    b_per_w = b // _NW
    n_chunks = b_per_w // _CHUNK
    assert n_chunks % 2 == 0
    mesh = plsc.VectorSubcoreMesh(
        core_axis_name="c", subcore_axis_name="s",
        num_cores=_NUM_CORES, num_subcores=_NUM_SUBCORES)
    out = pl.kernel(
        functools.partial(_emb_body, n_chunks, b_per_w),
        out_type=jax.ShapeDtypeStruct((b, D_MODEL), jnp.float32),
        mesh=mesh,
        scratch_types=[
            pltpu.VMEM((_CHUNK,), jnp.int32),
            pltpu.VMEM((_CHUNK,), jnp.int32),
            pltpu.VMEM((_CHUNK, D_MODEL), jnp.float32),
            pltpu.VMEM((_CHUNK, D_MODEL), jnp.float32),
            pltpu.SemaphoreType.DMA,
            pltpu.SemaphoreType.DMA,
            pltpu.SemaphoreType.DMA,
            pltpu.SemaphoreType.DMA,
        ],
        compiler_params=pltpu.CompilerParams(use_tc_tiling_on_sc=False),
    )(xf, table)
    return out.reshape(s0, s1, D_MODEL)
